# baseline (speedup 1.0000x reference)
import functools

import jax
import jax.numpy as jnp
from jax import lax
from jax.experimental import pallas as pl
from jax.experimental.pallas import tpu as pltpu
from jax.experimental.pallas import tpu_sc as plsc

_BATCH = 16384
_HIST = 20
_DIM = 64
_VOCAB = 1000000

_info = plsc.get_sparse_core_info()
_NC, _NS = _info.num_cores, _info.num_subcores
_NW = _NC * _NS

_VR = 384
_NCH1 = _VOCAB // _VR // 2 * 2
assert _VR * 2604 == 999936
_NCH1 = 2604
_IT1 = (_NCH1 + _NW - 1) // _NW
_TAIL0 = 999936

_mesh = plsc.VectorSubcoreMesh(core_axis_name="c", subcore_axis_name="s")


@functools.partial(
    pl.kernel,
    mesh=_mesh,
    out_type=jax.ShapeDtypeStruct((_VOCAB // 2, 128), jnp.float32),
    compiler_params=pltpu.CompilerParams(use_tc_tiling_on_sc=True, needs_layout_passes=False),
    scratch_types=[
        pltpu.VMEM((2, _DIM, _VR), jnp.float32),
        pltpu.VMEM((2, _VR // 2, 128), jnp.float32),
        pltpu.VMEM((32, 128), jnp.float32),
        pltpu.SemaphoreType.DMA((2,)),
        pltpu.SemaphoreType.DMA((2,)),
    ],
)
def _transpose_table(tt_hbm, tail_hbm, trm_hbm, bin_, bout, tailv, sem_i, sem_o):
    wid = lax.axis_index("s") * _NC + lax.axis_index("c")
    iota = lax.iota(jnp.int32, 16)
    dvecs = [iota + d0 for d0 in (0, 16, 32, 48)]

    def stage(i):
        c = wid + _NW * i
        b = i % 2
        return pltpu.async_copy(
            tt_hbm.at[:, pl.ds(c * _VR, _VR)], bin_.at[b], sem_i.at[b]
        )

    def writeout(i):
        c = wid + _NW * i
        b = i % 2
        return pltpu.async_copy(
            bout.at[b], trm_hbm.at[pl.ds(c * (_VR // 2), _VR // 2)], sem_o.at[b]
        )

    stage(0)
    stage(1)

    def chunk_body(i, carry):
        c = wid + _NW * i
        b = i % 2

        @pl.when(c < _NCH1)
        def _():
            pltpu.make_async_copy(
                tt_hbm.at[:, pl.ds(c * _VR, _VR)], bin_.at[b], sem_i.at[b]
            ).wait()

        @pl.when(jnp.logical_and(i >= 2, c - 2 * _NW < _NCH1))
        def _():
            pltpu.make_async_copy(
                bout.at[b],
                trm_hbm.at[pl.ds((c - 2 * _NW) * (_VR // 2), _VR // 2)],
                sem_o.at[b],
            ).wait()

        @pl.when(c < _NCH1)
        def _():
            def tr_body(v, carry2):
                vs = jnp.full((16,), v, dtype=jnp.int32)
                r = v >> 1
                col0 = (v & 1) * 64
                for k in range(4):
                    vals = plsc.load_gather(bin_.at[b], [dvecs[k], vs])
                    bout[b, r, pl.ds(col0 + k * 16, 16)] = vals
                return carry2

            lax.fori_loop(0, _VR, tr_body, 0)
            writeout(i)

        @pl.when(c + 2 * _NW < _NCH1)
        def _():
            stage(i + 2)

        return carry

    lax.fori_loop(0, _IT1, chunk_body, 0)

    def drain(i, carry):
        c = wid + _NW * i
        b = i % 2

        @pl.when(c < _NCH1)
        def _():
            pltpu.make_async_copy(
                bout.at[b],
                trm_hbm.at[pl.ds(c * (_VR // 2), _VR // 2)],
                sem_o.at[b],
            ).wait()

        return carry

    lax.fori_loop(_IT1 - 2, _IT1, drain, 0)

    @pl.when(wid == _NW - 1)
    def _():
        pltpu.sync_copy(tail_hbm, tailv)
        pltpu.sync_copy(tailv, trm_hbm.at[pl.ds(_TAIL0 // 2, 32)])


_BPW = _BATCH // _NW
_CH = 256
_NCH2 = _HIST * (_BPW // _CH)


@functools.partial(
    pl.kernel,
    mesh=_mesh,
    out_type=jax.ShapeDtypeStruct((_HIST, _DIM, _BATCH), jnp.float32),
    compiler_params=pltpu.CompilerParams(use_tc_tiling_on_sc=True, needs_layout_passes=False),
    scratch_types=[
        pltpu.VMEM((_HIST, _BPW), jnp.int32),
        pltpu.VMEM((2, 1, _CH), jnp.int32),
        pltpu.VMEM((2, _CH), jnp.int32),
        pltpu.VMEM((2, _CH, 128), jnp.float32),
        pltpu.VMEM((2, _DIM, _CH), jnp.float32),
        pltpu.SemaphoreType.DMA((2,)),
        pltpu.SemaphoreType.DMA((2,)),
    ],
)
def _gather_rows(xt_hbm, trm_hbm, out_hbm, idx_v, idx2, lsb64, rows, rows_t,
                 sem_g, sem_o):
    wid = lax.axis_index("s") * _NC + lax.axis_index("c")
    b0 = wid * _BPW
    pltpu.sync_copy(xt_hbm.at[:, pl.ds(b0, _BPW)], idx_v)
    iota = lax.iota(jnp.int32, 16)

    halves = _BPW // _CH

    def prep_and_gather(t):
        h = t // halves
        half = t % halves
        b = t % 2

        def split_body(j, carry):
            iv = idx_v[h, pl.ds(half * _CH + j * 16, 16)]
            idx2[b, 0, pl.ds(j * 16, 16)] = lax.shift_right_logical(iv, 1)
            lsb64[b, pl.ds(j * 16, 16)] = (iv & 1) * 64
            return carry

        lax.fori_loop(0, _CH // 16, split_body, 0)
        return pltpu.async_copy(trm_hbm.at[idx2.at[b, 0]], rows.at[b], sem_g.at[b])

    def chunk_body(t, carry):
        h = t // halves
        half = t % halves
        b = t % 2
        pltpu.make_async_copy(
            trm_hbm.at[idx2.at[b, 0]], rows.at[b], sem_g.at[b]
        ).wait()

        @pl.when(t >= 2)
        def _():
            pltpu.make_async_copy(
                rows_t.at[b],
                out_hbm.at[0, :, pl.ds(0, _CH)],
                sem_o.at[b],
            ).wait()

        def tr_body(jg, carry2):
            j0 = jg * 16
            jvec = iota + j0
            l64 = lsb64[b, pl.ds(j0, 16)]
            for d in range(_DIM):
                vals = plsc.load_gather(rows.at[b], [jvec, l64 + d])
                rows_t[b, d, pl.ds(j0, 16)] = vals
            return carry2

        lax.fori_loop(0, _CH // 16, tr_body, 0)
        pltpu.async_copy(
            rows_t.at[b],
            out_hbm.at[h, :, pl.ds(b0 + half * _CH, _CH)],
            sem_o.at[b],
        )

        @pl.when(t + 2 < _NCH2)
        def _():
            prep_and_gather(t + 2)

        return carry

    prep_and_gather(0)
    prep_and_gather(1)
    lax.fori_loop(0, _NCH2, chunk_body, 0)

    def drain(t, carry):
        b = t % 2
        h = t // halves
        half = t % halves
        pltpu.make_async_copy(
            rows_t.at[b],
            out_hbm.at[h, :, pl.ds(b0 + half * _CH, _CH)],
            sem_o.at[b],
        ).wait()
        return carry

    lax.fori_loop(_NCH2 - 2, _NCH2, drain, 0)


def kernel(x, table):
    tt = table.T
    tail = table[_TAIL0:, :].reshape(32, 128)
    trm = _transpose_table(tt, tail)
    out_t = _gather_rows(x.T, trm)
    return out_t.transpose(2, 0, 1)
